# Initial kernel scaffold; baseline (speedup 1.0000x reference)
import jax
import jax.numpy as jnp
from jax.experimental import pallas as pl


def kernel(mk, qk, mv):
    raise NotImplementedError("write your pallas kernel here")


# speedup vs baseline: 10.5507x; 10.5507x over previous
import math

import jax
import jax.numpy as jnp
from jax import lax
from jax.experimental import pallas as pl
from jax.experimental.pallas import tpu as pltpu

TOPK = 50
QT = 128

_SIGN = -2147483648


def _f32_to_ordered_i32(x):
    b = lax.bitcast_convert_type(x, jnp.int32)
    sign = jnp.full_like(b, _SIGN)
    return jnp.where(b < 0, jnp.bitwise_xor(jnp.bitwise_not(b), sign), b)


def _ordered_i32_to_f32(k):
    sign = jnp.full_like(k, _SIGN)
    b = jnp.where(k < 0, jnp.bitwise_not(jnp.bitwise_xor(k, sign)), k)
    return lax.bitcast_convert_type(b, jnp.float32)


def _body(mkT_ref, qk_ref, mv_ref, out_ref, key_ref):
    mkT = mkT_ref[...]
    ck = mkT.shape[1]
    a = jnp.sum(mkT * mkT, axis=1, keepdims=True)
    qk = qk_ref[...]
    s = jnp.dot(mkT, qk, preferred_element_type=jnp.float32)
    s = (2.0 * s - a) * (1.0 / math.sqrt(ck))
    smax = jnp.max(s, axis=0, keepdims=True)
    key_ref[...] = _f32_to_ordered_i32(s)

    lo = jnp.min(key_ref[...], axis=0, keepdims=True)
    hi = jnp.max(key_ref[...], axis=0, keepdims=True)

    def srch(_, carry):
        lo, hi = carry
        half = lax.shift_right_logical(hi - lo, 1)
        mid = lo + half
        cnt = jnp.sum((key_ref[...] >= mid).astype(jnp.int32), axis=0,
                      keepdims=True)
        pred = cnt >= TOPK
        return jnp.where(pred, mid, lo), jnp.where(pred, hi, mid - 1)

    lo, hi = lax.fori_loop(0, 33, srch, (lo, hi))
    t = lo

    keys = key_ref[...]
    gt = keys > t
    eq = keys == t
    cnt_gt = jnp.sum(gt.astype(jnp.int32), axis=0, keepdims=True)
    cnt_eq = jnp.sum(eq.astype(jnp.int32), axis=0, keepdims=True)
    frac = (TOPK - cnt_gt).astype(jnp.float32) / cnt_eq.astype(jnp.float32)
    sel = jnp.where(gt, 1.0, jnp.where(eq, frac, 0.0))
    sv = _ordered_i32_to_f32(keys)
    w = sel * jnp.exp(sv - smax)
    d = jnp.sum(w, axis=0, keepdims=True)
    wn = (w / d).astype(jnp.bfloat16)
    mv = mv_ref[...]
    out_ref[...] = jnp.dot(mv, wn, preferred_element_type=jnp.float32)


def kernel(mk, qk, mv):
    B, CK, T, H, W = mk.shape
    CV = mv.shape[1]
    M = T * H * W
    HW = H * W
    mkT = mk.reshape(CK, M).T
    qk_f = qk.reshape(CK, HW)
    mv_f = mv.reshape(CV, M).astype(jnp.bfloat16)

    out = pl.pallas_call(
        _body,
        grid=(HW // QT,),
        in_specs=[
            pl.BlockSpec((M, CK), lambda i: (0, 0)),
            pl.BlockSpec((CK, QT), lambda i: (0, i)),
            pl.BlockSpec((CV, M), lambda i: (0, 0)),
        ],
        out_specs=pl.BlockSpec((CV, QT), lambda i: (0, i)),
        out_shape=jax.ShapeDtypeStruct((CV, HW), jnp.float32),
        scratch_shapes=[pltpu.VMEM((M, QT), jnp.int32)],
    )(mkT, qk_f, mv_f)
    return out.reshape(B, CV, H, W)


# speedup vs baseline: 10.8403x; 1.0275x over previous
import jax
import jax.numpy as jnp
from jax import lax
from jax.experimental import pallas as pl

TOPK = 50.0
QT = 128

_SIGN = -2147483648


def _f2k(x):
    b = lax.bitcast_convert_type(x, jnp.int32)
    sign = jnp.full_like(b, _SIGN)
    return jnp.where(b < 0, jnp.bitwise_xor(jnp.bitwise_not(b), sign), b)


def _k2f(k):
    sign = jnp.full_like(k, _SIGN)
    b = jnp.where(k < 0, jnp.bitwise_not(jnp.bitwise_xor(k, sign)), k)
    return lax.bitcast_convert_type(b, jnp.float32)


def _body(mkT_ref, qks_ref, mv_ref, out_ref):
    mkT = mkT_ref[...]
    a = jnp.sum(mkT * mkT, axis=1, keepdims=True) * 0.125
    s = jnp.dot(mkT, qks_ref[...], preferred_element_type=jnp.float32) - a
    m = s.shape[0]
    smax = jnp.max(s, axis=0, keepdims=True)
    smin = jnp.min(s, axis=0, keepdims=True)

    klo0 = _f2k(smin)
    khi0 = _f2k(smax) + 1
    cl0 = jnp.full_like(smin, float(m))
    ch0 = jnp.zeros_like(smin)
    done0 = jnp.zeros(smin.shape, dtype=jnp.int32)

    def body(it, c):
        klo, khi, done, theta = c
        notdone = done == 0
        mid = klo + lax.shift_right_logical(khi - klo, 1)
        mid = jnp.maximum(mid, klo + 1)
        mid_f = _k2f(mid)
        cnt = jnp.sum(jnp.where(s >= mid_f, 1.0, 0.0), axis=0, keepdims=True)
        newly = jnp.logical_and(notdone, cnt == TOPK)
        theta = jnp.where(newly, mid_f, theta)
        wide = jnp.logical_and(khi != klo, khi != klo + 1)
        act = jnp.logical_and(jnp.logical_and(notdone, ~newly), wide)
        done = jnp.where(newly, 1, done)
        pred = cnt >= TOPK
        klo = jnp.where(jnp.logical_and(act, pred), mid, klo)
        khi = jnp.where(jnp.logical_and(act, ~pred), mid, khi)
        return klo, khi, done, theta

    init = (klo0, khi0, done0, smax)
    klo, khi, done, theta = lax.fori_loop(0, 33, body, init)
    t = jnp.where(done != 0, theta, _k2f(klo))

    gt = s > t
    ge = s >= t
    cnt_gt = jnp.sum(jnp.where(gt, 1.0, 0.0), axis=0, keepdims=True)
    cnt_ge = jnp.sum(jnp.where(ge, 1.0, 0.0), axis=0, keepdims=True)
    frac = (TOPK - cnt_gt) / (cnt_ge - cnt_gt)
    sel = jnp.where(gt, 1.0, jnp.where(ge, frac, 0.0))
    w = (sel * jnp.exp(s - smax)).astype(jnp.bfloat16)
    r = jnp.dot(mv_ref[...], w, preferred_element_type=jnp.float32)
    cv = r.shape[0] - 8
    out_ref[...] = r[:cv] / r[cv:cv + 1]


def kernel(mk, qk, mv):
    B, CK, T, H, W = mk.shape
    CV = mv.shape[1]
    M = T * H * W
    HW = H * W
    mkT = mk.reshape(CK, M).T
    qks = qk.reshape(CK, HW) * 0.25
    mv_aug = jnp.concatenate(
        [mv.reshape(CV, M),
         jnp.ones((1, M), jnp.float32),
         jnp.zeros((7, M), jnp.float32)], axis=0).astype(jnp.bfloat16)

    out = pl.pallas_call(
        _body,
        grid=(HW // QT,),
        in_specs=[
            pl.BlockSpec((M, CK), lambda i: (0, 0)),
            pl.BlockSpec((CK, QT), lambda i: (0, i)),
            pl.BlockSpec((CV + 8, M), lambda i: (0, 0)),
        ],
        out_specs=pl.BlockSpec((CV, QT), lambda i: (0, i)),
        out_shape=jax.ShapeDtypeStruct((CV, HW), jnp.float32),
    )(mkT, qks, mv_aug)
    return out.reshape(B, CV, H, W)


# speedup vs baseline: 15.1559x; 1.3981x over previous
import jax
import jax.numpy as jnp
from jax import lax
from jax.experimental import pallas as pl
from jax.experimental.pallas import tpu as pltpu

TOPK = 50.0
QT = 128

_SIGN = -2147483648


def _f2k(x):
    b = lax.bitcast_convert_type(x, jnp.int32)
    sign = jnp.full_like(b, _SIGN)
    return jnp.where(b < 0, jnp.bitwise_xor(jnp.bitwise_not(b), sign), b)


def _k2f(k):
    sign = jnp.full_like(k, _SIGN)
    b = jnp.where(k < 0, jnp.bitwise_not(jnp.bitwise_xor(k, sign)), k)
    return lax.bitcast_convert_type(b, jnp.float32)


def _body(mkT_ref, qks_ref, mv_ref, out_ref, sti_ref, stf_ref):
    mkT = mkT_ref[...]
    a = jnp.sum(mkT * mkT, axis=1, keepdims=True) * 0.125
    s = jnp.dot(mkT, qks_ref[...], preferred_element_type=jnp.float32) - a
    m = s.shape[0]
    smax = jnp.max(s, axis=0, keepdims=True)
    smin = jnp.min(s, axis=0, keepdims=True)

    sti_ref[0:1, :] = _f2k(smin)
    sti_ref[1:2, :] = _f2k(smax) + 1
    sti_ref[2:3, :] = jnp.zeros(smin.shape, jnp.int32)
    stf_ref[0:1, :] = jnp.full_like(smin, float(m))
    stf_ref[1:2, :] = jnp.zeros_like(smin)
    stf_ref[2:3, :] = smax

    def cond(c):
        it, nact = c
        return jnp.logical_and(it < 70, nact > 0)

    def body(c):
        it, _ = c
        klo = sti_ref[0:1, :]
        khi = sti_ref[1:2, :]
        done = sti_ref[2:3, :]
        cl = stf_ref[0:1, :]
        ch = stf_ref[1:2, :]
        notdone = done == 0
        flo = _k2f(klo)
        fhi = _k2f(khi)
        ti = flo + (cl - TOPK) / (cl - ch) * (fhi - flo)
        mid_interp = _f2k(ti)
        mid_bis = klo + lax.shift_right_logical(khi - klo, 1)
        mid = jnp.where((it % 2) == 0, mid_interp, mid_bis)
        mid = jnp.clip(mid, klo + 1, khi - 1)
        mid_f = _k2f(mid)
        cnt = jnp.sum(jnp.where(s >= mid_f, 1.0, 0.0), axis=0, keepdims=True)
        newly = jnp.logical_and(notdone, cnt == TOPK)
        stf_ref[2:3, :] = jnp.where(newly, mid_f, stf_ref[2:3, :])
        wide = jnp.logical_and(khi != klo, khi != klo + 1)
        act = jnp.logical_and(jnp.logical_and(notdone, ~newly), wide)
        done = jnp.where(newly, 1, done)
        sti_ref[2:3, :] = done
        pred = cnt >= TOPK
        up = jnp.logical_and(act, pred)
        dn = jnp.logical_and(act, ~pred)
        klo = jnp.where(up, mid, klo)
        khi = jnp.where(dn, mid, khi)
        sti_ref[0:1, :] = klo
        sti_ref[1:2, :] = khi
        stf_ref[0:1, :] = jnp.where(up, cnt, cl)
        stf_ref[1:2, :] = jnp.where(dn, cnt, ch)
        wide2 = jnp.logical_and(khi != klo, khi != klo + 1)
        still = jnp.logical_and(done == 0, wide2)
        nact = jnp.sum(still.astype(jnp.int32))
        return it + 1, nact

    lax.while_loop(cond, body, (jnp.int32(0), jnp.int32(1)))
    done = sti_ref[2:3, :]
    t = jnp.where(done != 0, stf_ref[2:3, :], _k2f(sti_ref[0:1, :]))

    gt = s > t
    ge = s >= t
    cnt_gt = jnp.sum(jnp.where(gt, 1.0, 0.0), axis=0, keepdims=True)
    cnt_ge = jnp.sum(jnp.where(ge, 1.0, 0.0), axis=0, keepdims=True)
    frac = (TOPK - cnt_gt) / (cnt_ge - cnt_gt)
    sel = jnp.where(gt, 1.0, jnp.where(ge, frac, 0.0))
    w = (sel * jnp.exp(s - smax)).astype(jnp.bfloat16)
    r = jnp.dot(mv_ref[...], w, preferred_element_type=jnp.float32)
    cv = r.shape[0] - 8
    out_ref[...] = r[:cv] / r[cv:cv + 1]


def kernel(mk, qk, mv):
    B, CK, T, H, W = mk.shape
    CV = mv.shape[1]
    M = T * H * W
    HW = H * W
    mkT = mk.reshape(CK, M).T
    qks = qk.reshape(CK, HW) * 0.25
    mv_aug = jnp.concatenate(
        [mv.reshape(CV, M),
         jnp.ones((1, M), jnp.float32),
         jnp.zeros((7, M), jnp.float32)], axis=0).astype(jnp.bfloat16)

    out = pl.pallas_call(
        _body,
        grid=(HW // QT,),
        in_specs=[
            pl.BlockSpec((M, CK), lambda i: (0, 0)),
            pl.BlockSpec((CK, QT), lambda i: (0, i)),
            pl.BlockSpec((CV + 8, M), lambda i: (0, 0)),
        ],
        out_specs=pl.BlockSpec((CV, QT), lambda i: (0, i)),
        out_shape=jax.ShapeDtypeStruct((CV, HW), jnp.float32),
        scratch_shapes=[pltpu.VMEM((8, QT), jnp.int32),
                        pltpu.VMEM((8, QT), jnp.float32)],
    )(mkT, qks, mv_aug)
    return out.reshape(B, CV, H, W)


# speedup vs baseline: 30.0809x; 1.9848x over previous
import jax
import jax.numpy as jnp
from jax import lax
from jax.experimental import pallas as pl
from jax.experimental.pallas import tpu as pltpu

TOPK = 50.0
QT = 128

_SIGN = -2147483648


def _f2k(x):
    b = lax.bitcast_convert_type(x, jnp.int32)
    sign = jnp.full_like(b, _SIGN)
    return jnp.where(b < 0, jnp.bitwise_xor(jnp.bitwise_not(b), sign), b)


def _k2f(k):
    sign = jnp.full_like(k, _SIGN)
    b = jnp.where(k < 0, jnp.bitwise_not(jnp.bitwise_xor(k, sign)), k)
    return lax.bitcast_convert_type(b, jnp.float32)


def _body(mkT_ref, qks_ref, mv_ref, out_ref, sti_ref, stf_ref):
    mkT = mkT_ref[...]
    a = jnp.sum(mkT * mkT, axis=1, keepdims=True) * 0.125
    s = jnp.dot(mkT, qks_ref[...], preferred_element_type=jnp.float32) - a
    m = s.shape[0]
    smax = jnp.max(s, axis=0, keepdims=True)
    smin = jnp.min(s, axis=0, keepdims=True)

    sti_ref[0:1, :] = _f2k(smin)
    sti_ref[1:2, :] = _f2k(smax) + 1
    sti_ref[2:3, :] = jnp.zeros(smin.shape, jnp.int32)
    stf_ref[0:1, :] = jnp.full_like(smin, float(m))
    stf_ref[1:2, :] = jnp.zeros_like(smin)
    stf_ref[2:3, :] = smax

    ones8 = jnp.ones((8, m), jnp.bfloat16)

    def cond(c):
        it, nact = c
        return jnp.logical_and(it < 70, nact > 0)

    def body(c):
        it, _ = c
        klo = sti_ref[0:1, :]
        khi = sti_ref[1:2, :]
        done = sti_ref[2:3, :]
        cl = stf_ref[0:1, :]
        ch = stf_ref[1:2, :]
        notdone = done == 0
        flo = _k2f(klo)
        fhi = _k2f(khi)
        lcl = jnp.log(cl)
        lch = jnp.log(jnp.maximum(ch, 0.5))
        ti = flo + (fhi - flo) * (lcl - 3.9120230054281460) / (lcl - lch)
        mid_interp = _f2k(ti)
        mid_bis = klo + lax.shift_right_logical(khi - klo, 1)
        mid = jnp.where((it % 2) == 0, mid_interp, mid_bis)
        mid = jnp.clip(mid, klo + 1, khi - 1)
        mid_f = _k2f(mid)
        ind = jnp.where(s >= mid_f, 1.0, 0.0).astype(jnp.bfloat16)
        cnt = jnp.dot(ones8, ind, preferred_element_type=jnp.float32)[0:1, :]
        newly = jnp.logical_and(notdone, cnt == TOPK)
        stf_ref[2:3, :] = jnp.where(newly, mid_f, stf_ref[2:3, :])
        wide = jnp.logical_and(khi != klo, khi != klo + 1)
        act = jnp.logical_and(jnp.logical_and(notdone, ~newly), wide)
        done = jnp.where(newly, 1, done)
        sti_ref[2:3, :] = done
        pred = cnt >= TOPK
        up = jnp.logical_and(act, pred)
        dn = jnp.logical_and(act, ~pred)
        klo = jnp.where(up, mid, klo)
        khi = jnp.where(dn, mid, khi)
        sti_ref[0:1, :] = klo
        sti_ref[1:2, :] = khi
        stf_ref[0:1, :] = jnp.where(up, cnt, cl)
        stf_ref[1:2, :] = jnp.where(dn, cnt, ch)
        wide2 = jnp.logical_and(khi != klo, khi != klo + 1)
        still = jnp.logical_and(done == 0, wide2)
        nact = jnp.sum(still.astype(jnp.int32))
        return it + 1, nact

    lax.while_loop(cond, body, (jnp.int32(0), jnp.int32(1)))
    done = sti_ref[2:3, :]
    t = jnp.where(done != 0, stf_ref[2:3, :], _k2f(sti_ref[0:1, :]))
    cl = stf_ref[0:1, :]
    ch = stf_ref[1:2, :]
    frac = jnp.where(done != 0, 1.0, (TOPK - ch) / (cl - ch))
    e = jnp.exp(s - smax)
    w = jnp.where(s > t, e,
                  jnp.where(s == t, frac * e, 0.0)).astype(jnp.bfloat16)
    r = jnp.dot(mv_ref[...], w, preferred_element_type=jnp.float32)
    cv = r.shape[0] - 8
    out_ref[...] = r[:cv] / r[cv:cv + 1]


def kernel(mk, qk, mv):
    B, CK, T, H, W = mk.shape
    CV = mv.shape[1]
    M = T * H * W
    HW = H * W
    mkT = mk.reshape(CK, M).T
    qks = qk.reshape(CK, HW) * 0.25
    mv_aug = jnp.concatenate(
        [mv.reshape(CV, M),
         jnp.ones((1, M), jnp.float32),
         jnp.zeros((7, M), jnp.float32)], axis=0).astype(jnp.bfloat16)

    out = pl.pallas_call(
        _body,
        grid=(HW // QT,),
        in_specs=[
            pl.BlockSpec((M, CK), lambda i: (0, 0)),
            pl.BlockSpec((CK, QT), lambda i: (0, i)),
            pl.BlockSpec((CV + 8, M), lambda i: (0, 0)),
        ],
        out_specs=pl.BlockSpec((CV, QT), lambda i: (0, i)),
        out_shape=jax.ShapeDtypeStruct((CV, HW), jnp.float32),
        scratch_shapes=[pltpu.VMEM((8, QT), jnp.int32),
                        pltpu.VMEM((8, QT), jnp.float32)],
    )(mkT, qks, mv_aug)
    return out.reshape(B, CV, H, W)


# speedup vs baseline: 30.4312x; 1.0116x over previous
import jax
import jax.numpy as jnp
from jax import lax
from jax.experimental import pallas as pl
from jax.experimental.pallas import tpu as pltpu

TOPK = 50.0
QT = 128

_SIGN = -2147483648


def _f2k(x):
    b = lax.bitcast_convert_type(x, jnp.int32)
    sign = jnp.full_like(b, _SIGN)
    return jnp.where(b < 0, jnp.bitwise_xor(jnp.bitwise_not(b), sign), b)


def _k2f(k):
    sign = jnp.full_like(k, _SIGN)
    b = jnp.where(k < 0, jnp.bitwise_not(jnp.bitwise_xor(k, sign)), k)
    return lax.bitcast_convert_type(b, jnp.float32)


def _body(mkT_ref, qks_ref, mv_ref, out_ref, sti_ref, stf_ref):
    mkT = mkT_ref[...]
    a = jnp.sum(mkT * mkT, axis=1, keepdims=True) * 0.125
    s = jnp.dot(mkT, qks_ref[...], preferred_element_type=jnp.float32) - a
    m = s.shape[0]
    smax = jnp.max(s, axis=0, keepdims=True)
    smin = jnp.min(s, axis=0, keepdims=True)

    sti_ref[0:1, :] = _f2k(smin)
    sti_ref[1:2, :] = _f2k(smax) + 1
    sti_ref[2:3, :] = jnp.zeros(smin.shape, jnp.int32)
    stf_ref[0:1, :] = jnp.full_like(smin, float(m))
    stf_ref[1:2, :] = jnp.zeros_like(smin)
    stf_ref[2:3, :] = smax

    ones8 = jnp.ones((8, m), jnp.bfloat16)

    def cond(c):
        it, nact = c
        return jnp.logical_and(it < 70, nact > 0)

    def body(c):
        it, _ = c
        klo = sti_ref[0:1, :]
        khi = sti_ref[1:2, :]
        done = sti_ref[2:3, :]
        cl = stf_ref[0:1, :]
        ch = stf_ref[1:2, :]
        notdone = done == 0
        flo = _k2f(klo)
        fhi = _k2f(khi)
        lcl = jnp.log(cl)
        lch = jnp.log(jnp.maximum(ch, 0.5))
        ti = flo + (fhi - flo) * (lcl - 3.9120230054281460) / (lcl - lch)
        mid_interp = _f2k(ti)
        mid_bis = klo + lax.shift_right_logical(khi - klo, 1)
        mid = jnp.where((it % 2) == 0, mid_interp, mid_bis)
        mid = jnp.clip(mid, klo + 1, khi - 1)
        mid_f = _k2f(mid)
        ind = jnp.where(s >= mid_f, 1.0, 0.0).astype(jnp.bfloat16)
        cnt = jnp.dot(ones8, ind, preferred_element_type=jnp.float32)[0:1, :]
        newly = jnp.logical_and(notdone, cnt == TOPK)
        stf_ref[2:3, :] = jnp.where(newly, mid_f, stf_ref[2:3, :])
        wide = jnp.logical_and(khi != klo, khi != klo + 1)
        act = jnp.logical_and(jnp.logical_and(notdone, ~newly), wide)
        done = jnp.where(newly, 1, done)
        sti_ref[2:3, :] = done
        pred = cnt >= TOPK
        up = jnp.logical_and(act, pred)
        dn = jnp.logical_and(act, ~pred)
        klo = jnp.where(up, mid, klo)
        khi = jnp.where(dn, mid, khi)
        sti_ref[0:1, :] = klo
        sti_ref[1:2, :] = khi
        stf_ref[0:1, :] = jnp.where(up, cnt, cl)
        stf_ref[1:2, :] = jnp.where(dn, cnt, ch)
        wide2 = jnp.logical_and(khi != klo, khi != klo + 1)
        still = jnp.logical_and(done == 0, wide2)
        nact = jnp.sum(still.astype(jnp.int32))
        return it + 1, nact

    lax.while_loop(cond, body, (jnp.int32(0), jnp.int32(1)))
    done = sti_ref[2:3, :]
    t = jnp.where(done != 0, stf_ref[2:3, :], _k2f(sti_ref[0:1, :]))
    cl = stf_ref[0:1, :]
    ch = stf_ref[1:2, :]
    frac = jnp.where(done != 0, 1.0, (TOPK - ch) / (cl - ch))
    e = jnp.exp(s - smax)
    w = jnp.where(s > t, e,
                  jnp.where(s == t, frac * e, 0.0)).astype(jnp.bfloat16)
    d = jnp.dot(ones8, w, preferred_element_type=jnp.float32)[0:1, :]
    r = jnp.dot(mv_ref[...], w, preferred_element_type=jnp.float32)
    out_ref[...] = r / d


def kernel(mk, qk, mv):
    B, CK, T, H, W = mk.shape
    CV = mv.shape[1]
    M = T * H * W
    HW = H * W
    mkT = mk.reshape(CK, M).T
    qks = qk.reshape(CK, HW) * 0.25
    mv_b = mv.reshape(CV, M).astype(jnp.bfloat16)

    out = pl.pallas_call(
        _body,
        grid=(HW // QT,),
        in_specs=[
            pl.BlockSpec((M, CK), lambda i: (0, 0)),
            pl.BlockSpec((CK, QT), lambda i: (0, i)),
            pl.BlockSpec((CV, M), lambda i: (0, 0)),
        ],
        out_specs=pl.BlockSpec((CV, QT), lambda i: (0, i)),
        out_shape=jax.ShapeDtypeStruct((CV, HW), jnp.float32),
        scratch_shapes=[pltpu.VMEM((8, QT), jnp.int32),
                        pltpu.VMEM((8, QT), jnp.float32)],
    )(mkT, qks, mv_b)
    return out.reshape(B, CV, H, W)


# speedup vs baseline: 35.0074x; 1.1504x over previous
import jax
import jax.numpy as jnp
from jax import lax
from jax.experimental import pallas as pl
from jax.experimental.pallas import tpu as pltpu

TOPK = 50.0
QT = 128

_SIGN = -2147483648


def _f2k(x):
    b = lax.bitcast_convert_type(x, jnp.int32)
    sign = jnp.full_like(b, _SIGN)
    return jnp.where(b < 0, jnp.bitwise_xor(jnp.bitwise_not(b), sign), b)


def _k2f(k):
    sign = jnp.full_like(k, _SIGN)
    b = jnp.where(k < 0, jnp.bitwise_not(jnp.bitwise_xor(k, sign)), k)
    return lax.bitcast_convert_type(b, jnp.float32)


def _body(mkT_ref, qks_ref, mv_ref, out_ref, sti_ref, stf_ref):
    mkT = mkT_ref[...]
    a = jnp.sum(mkT * mkT, axis=1, keepdims=True) * 0.125
    s = jnp.dot(mkT, qks_ref[...], preferred_element_type=jnp.float32) - a
    m = s.shape[0]
    smax = jnp.max(s, axis=0, keepdims=True)
    smin = jnp.min(s, axis=0, keepdims=True)

    sti_ref[0:1, :] = _f2k(smin)
    sti_ref[1:2, :] = _f2k(smax) + 1
    sti_ref[2:3, :] = jnp.zeros(smin.shape, jnp.int32)
    stf_ref[0:1, :] = jnp.full_like(smin, float(m))
    stf_ref[1:2, :] = jnp.zeros_like(smin)
    stf_ref[2:3, :] = smax

    ones8 = jnp.ones((8, m), jnp.bfloat16)
    ones8f = jnp.ones((8, m), jnp.float32)
    ssum = jnp.dot(ones8f, s, preferred_element_type=jnp.float32)[0:1, :]
    s2sum = jnp.dot(ones8f, s * s, preferred_element_type=jnp.float32)[0:1, :]
    mu = ssum * (1.0 / m)
    sig = jnp.sqrt(jnp.maximum(s2sum * (1.0 / m) - mu * mu, 0.0))
    theta0 = mu + 2.814664864395869 * sig

    def cond(c):
        it, nact = c
        return jnp.logical_and(it < 140, nact > 0)

    def body(c):
        it, _ = c
        klo = sti_ref[0:1, :]
        khi = sti_ref[1:2, :]
        done = sti_ref[2:3, :]
        cl = stf_ref[0:1, :]
        ch = stf_ref[1:2, :]
        notdone = done == 0
        flo = _k2f(klo)
        fhi = _k2f(khi)
        lcl = jnp.log(cl)
        lch = jnp.log(jnp.maximum(ch, 0.5))
        ti = flo + (fhi - flo) * (lcl - 3.9120230054281460) / (lcl - lch)
        ti = jnp.where(it == 0, theta0, ti)
        mid_interp = _f2k(ti)
        mid_bis = klo + lax.shift_right_logical(khi - klo, 1)
        mid = jnp.where((it % 4) != 3, mid_interp, mid_bis)
        mid = jnp.clip(mid, klo + 1, khi - 1)
        mid_f = _k2f(mid)
        ind = jnp.where(s >= mid_f, 1.0, 0.0).astype(jnp.bfloat16)
        cnt = jnp.dot(ones8, ind, preferred_element_type=jnp.float32)[0:1, :]
        newly = jnp.logical_and(notdone, cnt == TOPK)
        stf_ref[2:3, :] = jnp.where(newly, mid_f, stf_ref[2:3, :])
        wide = jnp.logical_and(khi != klo, khi != klo + 1)
        act = jnp.logical_and(jnp.logical_and(notdone, ~newly), wide)
        done = jnp.where(newly, 1, done)
        sti_ref[2:3, :] = done
        pred = cnt >= TOPK
        up = jnp.logical_and(act, pred)
        dn = jnp.logical_and(act, ~pred)
        klo = jnp.where(up, mid, klo)
        khi = jnp.where(dn, mid, khi)
        sti_ref[0:1, :] = klo
        sti_ref[1:2, :] = khi
        stf_ref[0:1, :] = jnp.where(up, cnt, cl)
        stf_ref[1:2, :] = jnp.where(dn, cnt, ch)
        wide2 = jnp.logical_and(khi != klo, khi != klo + 1)
        still = jnp.logical_and(done == 0, wide2)
        nact = jnp.sum(still.astype(jnp.int32))
        return it + 1, nact

    lax.while_loop(cond, body, (jnp.int32(0), jnp.int32(1)))
    done = sti_ref[2:3, :]
    t = jnp.where(done != 0, stf_ref[2:3, :], _k2f(sti_ref[0:1, :]))
    cl = stf_ref[0:1, :]
    ch = stf_ref[1:2, :]
    frac = jnp.where(done != 0, 1.0, (TOPK - ch) / (cl - ch))
    e = jnp.exp(s - smax)
    w = jnp.where(s > t, e,
                  jnp.where(s == t, frac * e, 0.0)).astype(jnp.bfloat16)
    d = jnp.dot(ones8, w, preferred_element_type=jnp.float32)[0:1, :]
    r = jnp.dot(mv_ref[...], w, preferred_element_type=jnp.float32)
    out_ref[...] = r / d


def kernel(mk, qk, mv):
    B, CK, T, H, W = mk.shape
    CV = mv.shape[1]
    M = T * H * W
    HW = H * W
    mkT = mk.reshape(CK, M).T
    qks = qk.reshape(CK, HW) * 0.25
    mv_b = mv.reshape(CV, M).astype(jnp.bfloat16)

    out = pl.pallas_call(
        _body,
        grid=(HW // QT,),
        in_specs=[
            pl.BlockSpec((M, CK), lambda i: (0, 0)),
            pl.BlockSpec((CK, QT), lambda i: (0, i)),
            pl.BlockSpec((CV, M), lambda i: (0, 0)),
        ],
        out_specs=pl.BlockSpec((CV, QT), lambda i: (0, i)),
        out_shape=jax.ShapeDtypeStruct((CV, HW), jnp.float32),
        scratch_shapes=[pltpu.VMEM((8, QT), jnp.int32),
                        pltpu.VMEM((8, QT), jnp.float32)],
    )(mkT, qks, mv_b)
    return out.reshape(B, CV, H, W)


# speedup vs baseline: 38.5381x; 1.1009x over previous
import jax
import jax.numpy as jnp
from jax import lax
from jax.experimental import pallas as pl
from jax.experimental.pallas import tpu as pltpu

TOPK = 50.0
QT = 128

_SIGN = -2147483648


def _f2k(x):
    b = lax.bitcast_convert_type(x, jnp.int32)
    sign = jnp.full_like(b, _SIGN)
    return jnp.where(b < 0, jnp.bitwise_xor(jnp.bitwise_not(b), sign), b)


def _k2f(k):
    sign = jnp.full_like(k, _SIGN)
    b = jnp.where(k < 0, jnp.bitwise_not(jnp.bitwise_xor(k, sign)), k)
    return lax.bitcast_convert_type(b, jnp.float32)


def _zq(p):
    p = jnp.clip(p, 3e-5, 1.0 - 3e-5)
    pl = jnp.minimum(p, 1.0 - p)
    t = jnp.sqrt(-2.0 * jnp.log(pl))
    z = t - (2.515517 + t * (0.802853 + t * 0.010328)) / (
        1.0 + t * (1.432788 + t * (0.189269 + t * 0.001308)))
    return jnp.where(p <= 0.5, z, -z)


def _body(mkT_ref, qks_ref, mv_ref, out_ref, sti_ref, stf_ref):
    mkT = mkT_ref[...]
    a = jnp.sum(mkT * mkT, axis=1, keepdims=True) * 0.125
    s = jnp.dot(mkT, qks_ref[...], preferred_element_type=jnp.float32) - a
    m = s.shape[0]
    smax = jnp.max(s, axis=0, keepdims=True)
    smin = jnp.min(s, axis=0, keepdims=True)

    sti_ref[0:1, :] = _f2k(smin)
    sti_ref[1:2, :] = _f2k(smax) + 1
    sti_ref[2:3, :] = jnp.zeros(smin.shape, jnp.int32)
    stf_ref[0:1, :] = jnp.full_like(smin, float(m))
    stf_ref[1:2, :] = jnp.zeros_like(smin)
    stf_ref[2:3, :] = smax

    ones8 = jnp.ones((8, m), jnp.bfloat16)
    ones8f = jnp.ones((8, m), jnp.float32)
    ssum = jnp.dot(ones8f, s, preferred_element_type=jnp.float32)[0:1, :]
    s2sum = jnp.dot(ones8f, s * s, preferred_element_type=jnp.float32)[0:1, :]
    mu = ssum * (1.0 / m)
    sig = jnp.sqrt(jnp.maximum(s2sum * (1.0 / m) - mu * mu, 0.0))
    theta0 = mu + 2.814664864395869 * sig

    def cond(c):
        it, nact = c
        return jnp.logical_and(it < 280, nact > 0)

    def body(c):
        it, _ = c
        klo = sti_ref[0:1, :]
        khi = sti_ref[1:2, :]
        done = sti_ref[2:3, :]
        cl = stf_ref[0:1, :]
        ch = stf_ref[1:2, :]
        notdone = done == 0
        flo = _k2f(klo)
        fhi = _k2f(khi)
        inv_m = 1.0 / m
        zl = _zq(cl * inv_m)
        zh = _zq(jnp.maximum(ch, 0.5) * inv_m)
        ti = flo + (fhi - flo) * (zl - 2.815024087832663) / (zl - zh)
        ti = jnp.where(it == 0, theta0, ti)
        mid_interp = _f2k(ti)
        mid_bis = klo + lax.shift_right_logical(khi - klo, 1)
        mid = jnp.where((it % 8) != 7, mid_interp, mid_bis)
        mid = jnp.clip(mid, klo + 1, khi - 1)
        mid_f = _k2f(mid)
        ind = jnp.where(s >= mid_f, 1.0, 0.0).astype(jnp.bfloat16)
        cnt = jnp.dot(ones8, ind, preferred_element_type=jnp.float32)[0:1, :]
        newly = jnp.logical_and(notdone, cnt == TOPK)
        stf_ref[2:3, :] = jnp.where(newly, mid_f, stf_ref[2:3, :])
        wide = jnp.logical_and(khi != klo, khi != klo + 1)
        act = jnp.logical_and(jnp.logical_and(notdone, ~newly), wide)
        done = jnp.where(newly, 1, done)
        sti_ref[2:3, :] = done
        pred = cnt >= TOPK
        up = jnp.logical_and(act, pred)
        dn = jnp.logical_and(act, ~pred)
        klo = jnp.where(up, mid, klo)
        khi = jnp.where(dn, mid, khi)
        sti_ref[0:1, :] = klo
        sti_ref[1:2, :] = khi
        stf_ref[0:1, :] = jnp.where(up, cnt, cl)
        stf_ref[1:2, :] = jnp.where(dn, cnt, ch)
        wide2 = jnp.logical_and(khi != klo, khi != klo + 1)
        still = jnp.logical_and(done == 0, wide2)
        nact = jnp.sum(still.astype(jnp.int32))
        return it + 1, nact

    lax.while_loop(cond, body, (jnp.int32(0), jnp.int32(1)))
    done = sti_ref[2:3, :]
    t = jnp.where(done != 0, stf_ref[2:3, :], _k2f(sti_ref[0:1, :]))
    cl = stf_ref[0:1, :]
    ch = stf_ref[1:2, :]
    frac = jnp.where(done != 0, 1.0, (TOPK - ch) / (cl - ch))
    e = jnp.exp(s - smax)
    w = jnp.where(s > t, e,
                  jnp.where(s == t, frac * e, 0.0)).astype(jnp.bfloat16)
    d = jnp.dot(ones8, w, preferred_element_type=jnp.float32)[0:1, :]
    r = jnp.dot(mv_ref[...], w, preferred_element_type=jnp.float32)
    out_ref[...] = r / d


def kernel(mk, qk, mv):
    B, CK, T, H, W = mk.shape
    CV = mv.shape[1]
    M = T * H * W
    HW = H * W
    mkT = mk.reshape(CK, M).T
    qks = qk.reshape(CK, HW) * 0.25
    mv_b = mv.reshape(CV, M).astype(jnp.bfloat16)

    out = pl.pallas_call(
        _body,
        grid=(HW // QT,),
        in_specs=[
            pl.BlockSpec((M, CK), lambda i: (0, 0)),
            pl.BlockSpec((CK, QT), lambda i: (0, i)),
            pl.BlockSpec((CV, M), lambda i: (0, 0)),
        ],
        out_specs=pl.BlockSpec((CV, QT), lambda i: (0, i)),
        out_shape=jax.ShapeDtypeStruct((CV, HW), jnp.float32),
        scratch_shapes=[pltpu.VMEM((8, QT), jnp.int32),
                        pltpu.VMEM((8, QT), jnp.float32)],
    )(mkT, qks, mv_b)
    return out.reshape(B, CV, H, W)


# speedup vs baseline: 38.5555x; 1.0005x over previous
import jax
import jax.numpy as jnp
from jax import lax
from jax.experimental import pallas as pl
from jax.experimental.pallas import tpu as pltpu

TOPK = 50.0
QT = 128

_SIGN = -2147483648


def _f2k(x):
    b = lax.bitcast_convert_type(x, jnp.int32)
    sign = jnp.full_like(b, _SIGN)
    return jnp.where(b < 0, jnp.bitwise_xor(jnp.bitwise_not(b), sign), b)


def _k2f(k):
    sign = jnp.full_like(k, _SIGN)
    b = jnp.where(k < 0, jnp.bitwise_not(jnp.bitwise_xor(k, sign)), k)
    return lax.bitcast_convert_type(b, jnp.float32)


def _zq(p):
    p = jnp.clip(p, 3e-5, 1.0 - 3e-5)
    pl = jnp.minimum(p, 1.0 - p)
    t = jnp.sqrt(-2.0 * jnp.log(pl))
    z = t - (2.515517 + t * (0.802853 + t * 0.010328)) / (
        1.0 + t * (1.432788 + t * (0.189269 + t * 0.001308)))
    return jnp.where(p <= 0.5, z, -z)


def _body(mkT_ref, qks_ref, mv_ref, out_ref, sti_ref, stf_ref):
    mkT = mkT_ref[...]
    a = jnp.sum(mkT * mkT, axis=1, keepdims=True) * 0.125
    qks = qks_ref[...] * 0.25
    s = jnp.dot(mkT, qks, preferred_element_type=jnp.float32) - a
    m = s.shape[0]
    smax = jnp.max(s, axis=0, keepdims=True)
    smin = jnp.min(s, axis=0, keepdims=True)

    sti_ref[0:1, :] = _f2k(smin)
    sti_ref[1:2, :] = _f2k(smax) + 1
    sti_ref[2:3, :] = jnp.zeros(smin.shape, jnp.int32)
    stf_ref[0:1, :] = jnp.full_like(smin, float(m))
    stf_ref[1:2, :] = jnp.zeros_like(smin)
    stf_ref[2:3, :] = smax

    ones8 = jnp.ones((8, m), jnp.bfloat16)
    ones8f = jnp.ones((8, m), jnp.float32)
    ssum = jnp.dot(ones8f, s, preferred_element_type=jnp.float32)[0:1, :]
    s2sum = jnp.dot(ones8f, s * s, preferred_element_type=jnp.float32)[0:1, :]
    mu = ssum * (1.0 / m)
    sig = jnp.sqrt(jnp.maximum(s2sum * (1.0 / m) - mu * mu, 0.0))
    theta0 = mu + 2.814664864395869 * sig

    def cond(c):
        it, nact = c
        return jnp.logical_and(it < 280, nact > 0)

    def body(c):
        it, _ = c
        klo = sti_ref[0:1, :]
        khi = sti_ref[1:2, :]
        done = sti_ref[2:3, :]
        cl = stf_ref[0:1, :]
        ch = stf_ref[1:2, :]
        notdone = done == 0
        flo = _k2f(klo)
        fhi = _k2f(khi)
        inv_m = 1.0 / m
        zl = _zq(cl * inv_m)
        zh = _zq(jnp.maximum(ch, 0.5) * inv_m)
        ti = flo + (fhi - flo) * (zl - 2.815024087832663) / (zl - zh)
        ti = jnp.where(it == 0, theta0, ti)
        mid_interp = _f2k(ti)
        mid_bis = klo + lax.shift_right_logical(khi - klo, 1)
        mid = jnp.where((it % 8) != 7, mid_interp, mid_bis)
        mid = jnp.clip(mid, klo + 1, khi - 1)
        mid_f = _k2f(mid)
        ind = jnp.where(s >= mid_f, 1.0, 0.0).astype(jnp.bfloat16)
        cnt = jnp.dot(ones8, ind, preferred_element_type=jnp.float32)[0:1, :]
        newly = jnp.logical_and(notdone, cnt == TOPK)
        stf_ref[2:3, :] = jnp.where(newly, mid_f, stf_ref[2:3, :])
        wide = jnp.logical_and(khi != klo, khi != klo + 1)
        act = jnp.logical_and(jnp.logical_and(notdone, ~newly), wide)
        done = jnp.where(newly, 1, done)
        sti_ref[2:3, :] = done
        pred = cnt >= TOPK
        up = jnp.logical_and(act, pred)
        dn = jnp.logical_and(act, ~pred)
        klo = jnp.where(up, mid, klo)
        khi = jnp.where(dn, mid, khi)
        sti_ref[0:1, :] = klo
        sti_ref[1:2, :] = khi
        stf_ref[0:1, :] = jnp.where(up, cnt, cl)
        stf_ref[1:2, :] = jnp.where(dn, cnt, ch)
        wide2 = jnp.logical_and(khi != klo, khi != klo + 1)
        still = jnp.logical_and(done == 0, wide2)
        nact = jnp.sum(still.astype(jnp.int32))
        return it + 1, nact

    lax.while_loop(cond, body, (jnp.int32(0), jnp.int32(1)))
    done = sti_ref[2:3, :]
    t = jnp.where(done != 0, stf_ref[2:3, :], _k2f(sti_ref[0:1, :]))
    cl = stf_ref[0:1, :]
    ch = stf_ref[1:2, :]
    frac = jnp.where(done != 0, 1.0, (TOPK - ch) / (cl - ch))
    e = jnp.exp(s - smax)
    w = jnp.where(s > t, e,
                  jnp.where(s == t, frac * e, 0.0)).astype(jnp.bfloat16)
    d = jnp.dot(ones8, w, preferred_element_type=jnp.float32)[0:1, :]
    r = jnp.dot(mv_ref[...], w, preferred_element_type=jnp.float32)
    out_ref[...] = r / d


def kernel(mk, qk, mv):
    B, CK, T, H, W = mk.shape
    CV = mv.shape[1]
    M = T * H * W
    HW = H * W
    mkT = mk.reshape(CK, M).T
    qk_f = qk.reshape(CK, HW)
    mv_b = mv.reshape(CV, M).astype(jnp.bfloat16)

    out = pl.pallas_call(
        _body,
        grid=(HW // QT,),
        in_specs=[
            pl.BlockSpec((M, CK), lambda i: (0, 0)),
            pl.BlockSpec((CK, QT), lambda i: (0, i)),
            pl.BlockSpec((CV, M), lambda i: (0, 0)),
        ],
        out_specs=pl.BlockSpec((CV, QT), lambda i: (0, i)),
        out_shape=jax.ShapeDtypeStruct((CV, HW), jnp.float32),
        scratch_shapes=[pltpu.VMEM((8, QT), jnp.int32),
                        pltpu.VMEM((8, QT), jnp.float32)],
    )(mkT, qk_f, mv_b)
    return out.reshape(B, CV, H, W)
